# baseline (speedup 1.0000x reference)
import functools

import jax
import jax.numpy as jnp
from jax import lax
from jax.experimental import pallas as pl
from jax.experimental.pallas import tpu as pltpu
from jax.experimental.pallas import tpu_sc as plsc

DIM = 64
B = 16384
LAMBD = 0.1
NW = 32
CHUNK = B // NW
NSEG = 4
SEG = CHUNK // NSEG
GROUPS = CHUNK // 16

_mesh = plsc.VectorSubcoreMesh(core_axis_name="c", subcore_axis_name="s",
                               num_cores=2, num_subcores=16)


@functools.partial(
    pl.kernel,
    mesh=_mesh,
    compiler_params=pltpu.CompilerParams(needs_layout_passes=False,
                                         use_tc_tiling_on_sc=False),
    out_type=[
        jax.ShapeDtypeStruct((B,), jnp.float32),
        jax.ShapeDtypeStruct((B,), jnp.float32),
        jax.ShapeDtypeStruct((B,), jnp.float32),
        jax.ShapeDtypeStruct((B,), jnp.float32),
    ],
    scratch_types=[
        pltpu.VMEM((NSEG, SEG), jnp.int32),
        pltpu.VMEM((NSEG, SEG), jnp.int32),
        pltpu.VMEM((CHUNK, DIM), jnp.float32),
        pltpu.VMEM((CHUNK, DIM), jnp.float32),
        pltpu.VMEM((CHUNK,), jnp.float32),
        pltpu.VMEM((CHUNK,), jnp.float32),
        pltpu.VMEM((CHUNK,), jnp.float32),
        pltpu.SemaphoreType.DMA,
    ],
)
def _sc_gather_dot(src_hbm, ctx_hbm, pure_hbm, per_hbm,
                   node_hbm, noise_hbm, base_hbm,
                   s_out, na_out, nb_out, r_out,
                   idx_a, idx_b, rows_a, rows_b, s_v, na_v, nb_v, sem):
    wid = lax.axis_index("s") * 2 + lax.axis_index("c")
    base = wid * CHUNK

    def gather_pair(tab_a, tab_b):
        handles = []
        for k in range(NSEG):
            handles.append(pltpu.async_copy(
                tab_a.at[idx_a.at[k]], rows_a.at[pl.ds(k * SEG, SEG)], sem))
            handles.append(pltpu.async_copy(
                tab_b.at[idx_b.at[k]], rows_b.at[pl.ds(k * SEG, SEG)], sem))
        for h in handles:
            h.wait()

    pltpu.sync_copy(src_hbm.at[wid], idx_a)
    pltpu.sync_copy(ctx_hbm.at[wid], idx_b)
    gather_pair(node_hbm, noise_hbm)

    zero = jnp.zeros((16,), jnp.float32)

    def main_group(g, _):
        rows = g * 16 + lax.iota(jnp.int32, 16)

        def col(j, acc):
            s, na, nb = acc
            cols = jnp.full((16,), 0, jnp.int32) + j
            a = plsc.load_gather(rows_a, [rows, cols])
            b = plsc.load_gather(rows_b, [rows, cols])
            return (s + a * b, na + a * a, nb + b * b)

        s, na, nb = lax.fori_loop(0, DIM, col, (zero, zero, zero))
        s_v[pl.ds(g * 16, 16)] = s
        na_v[pl.ds(g * 16, 16)] = na
        nb_v[pl.ds(g * 16, 16)] = nb
        return 0

    lax.fori_loop(0, GROUPS, main_group, 0)
    pltpu.sync_copy(s_v, s_out.at[pl.ds(base, CHUNK)])
    pltpu.sync_copy(na_v, na_out.at[pl.ds(base, CHUNK)])
    pltpu.sync_copy(nb_v, nb_out.at[pl.ds(base, CHUNK)])

    pltpu.sync_copy(pure_hbm.at[wid], idx_a)
    pltpu.sync_copy(per_hbm.at[wid], idx_b)
    gather_pair(node_hbm, base_hbm)

    def reg_group(g, _):
        rows = g * 16 + lax.iota(jnp.int32, 16)

        def col(j, s):
            cols = jnp.full((16,), 0, jnp.int32) + j
            a = plsc.load_gather(rows_a, [rows, cols])
            b = plsc.load_gather(rows_b, [rows, cols])
            return s + a * b

        s = lax.fori_loop(0, DIM, col, zero)
        s_v[pl.ds(g * 16, 16)] = s
        return 0

    lax.fori_loop(0, GROUPS, reg_group, 0)
    pltpu.sync_copy(s_v, r_out.at[pl.ds(base, CHUNK)])


def _finish_body(t_ref, s_ref, na_ref, nb_ref, r_ref, o_ref):
    na = jnp.maximum(jnp.sqrt(na_ref[...]), 1e-12)
    nb = jnp.maximum(jnp.sqrt(nb_ref[...]), 1e-12)
    scores = jax.nn.sigmoid(s_ref[...] / (na * nb))
    t = t_ref[...]
    main = t * jnp.log(scores) + (1.0 - t) * jnp.log(1.0 - scores)
    main_loss = -jnp.mean(main)
    r = jax.nn.sigmoid(jnp.clip(r_ref[...], -15.0, 15.0))
    reg_loss = -jnp.mean(jnp.log(r))
    o_ref[...] = jnp.reshape(main_loss + LAMBD * reg_loss, (1, 1))


_finish = pl.pallas_call(
    _finish_body,
    out_shape=jax.ShapeDtypeStruct((1, 1), jnp.float32),
)


@jax.jit
def kernel(sources, contexts, targets, personas, pure_sources,
           node_embedding, node_noise_embedding, base_node_embedding):
    src = sources.astype(jnp.int32).reshape(NW, NSEG, SEG)
    ctx = contexts.astype(jnp.int32).reshape(NW, NSEG, SEG)
    pure = pure_sources.astype(jnp.int32).reshape(NW, NSEG, SEG)
    per = personas.astype(jnp.int32).reshape(NW, NSEG, SEG)
    s, na, nb, r = _sc_gather_dot(src, ctx, pure, per,
                                  node_embedding, node_noise_embedding,
                                  base_node_embedding)
    out = _finish(targets.reshape(128, 128), s.reshape(128, 128),
                  na.reshape(128, 128), nb.reshape(128, 128),
                  r.reshape(128, 128))
    return out.reshape(())
